# baseline (speedup 1.0000x reference)
import functools

import jax
import jax.numpy as jnp
from jax import lax
from jax.experimental import pallas as pl
from jax.experimental.pallas import tpu as pltpu
from jax.experimental.pallas import tpu_sc as plsc

BATCH = 16384
D_ID = 128
D_REVIEW = 64
CHUNK = 128


def _build_kernel():
    info = plsc.get_sparse_core_info()
    num_cores = info.num_cores
    num_workers = num_cores * info.num_subcores
    b_per_w = BATCH // num_workers
    n_chunks = b_per_w // CHUNK

    mesh = plsc.VectorSubcoreMesh(core_axis_name="c", subcore_axis_name="s")

    @functools.partial(
        pl.kernel,
        mesh=mesh,
        compiler_params=pltpu.CompilerParams(use_tc_tiling_on_sc=False),
        out_type=[
            jax.ShapeDtypeStruct((BATCH, D_ID), jnp.float32),
            jax.ShapeDtypeStruct((BATCH, D_ID), jnp.float32),
            jax.ShapeDtypeStruct((BATCH, D_REVIEW), jnp.float32),
        ],
        scratch_types=[
            pltpu.VMEM((n_chunks, CHUNK), jnp.int32),
            pltpu.VMEM((CHUNK, D_ID), jnp.float32),
            pltpu.VMEM((CHUNK, D_ID), jnp.float32),
            pltpu.VMEM((CHUNK, D_REVIEW), jnp.float32),
            pltpu.VMEM((CHUNK, D_ID), jnp.float32),
            pltpu.VMEM((CHUNK, D_ID), jnp.float32),
            pltpu.VMEM((CHUNK, D_REVIEW), jnp.float32),
            pltpu.SemaphoreType.DMA,
            pltpu.SemaphoreType.DMA,
        ],
    )
    def gather3(idx_hbm, protos_hbm, emb_hbm, review_hbm,
                proto_out, emb_out, review_out,
                idx_v, pv0, ev0, rv0, pv1, ev1, rv1, sem0, sem1):
        wid = lax.axis_index("s") * num_cores + lax.axis_index("c")
        base = wid * b_per_w
        pltpu.sync_copy(idx_hbm.at[wid], idx_v)

        slots = ((pv0, ev0, rv0, sem0), (pv1, ev1, rv1, sem1))

        def start(j, slot):
            pv, ev, rv, sem = slot
            row_idx = idx_v.at[j]
            return (
                pltpu.async_copy(protos_hbm.at[row_idx], pv, sem),
                pltpu.async_copy(emb_hbm.at[row_idx], ev, sem),
                pltpu.async_copy(review_hbm.at[row_idx], rv, sem),
            )

        handles = [None, None]
        handles[0] = start(0, slots[0])
        for j in range(n_chunks):
            if j + 1 < n_chunks:
                handles[(j + 1) % 2] = start(j + 1, slots[(j + 1) % 2])
            for h in handles[j % 2]:
                h.wait()
            pv, ev, rv, _ = slots[j % 2]
            off = base + j * CHUNK
            pltpu.sync_copy(pv, proto_out.at[pl.ds(off, CHUNK)])
            pltpu.sync_copy(ev, emb_out.at[pl.ds(off, CHUNK)])
            pltpu.sync_copy(rv, review_out.at[pl.ds(off, CHUNK)])

    return gather3, num_workers, n_chunks


def kernel(nodes_u, global_protos, u_emb_weight, u_review_weight):
    gather3, num_workers, n_chunks = _build_kernel()
    idx = nodes_u.astype(jnp.int32).reshape(num_workers, n_chunks, CHUNK)
    proto_feats, u_id_feats, u_review_feats = gather3(
        idx, global_protos, u_emb_weight, u_review_weight)
    return (proto_feats, u_id_feats, u_review_feats)
